# Initial kernel scaffold; baseline (speedup 1.0000x reference)
import jax
import jax.numpy as jnp
from jax.experimental import pallas as pl


def kernel(x, edge_index, edge_attr, batch, W_enc, Wg0, We0, bg0, Wg1, We1, bg1, Wg2, We2, bg2, p0, p1, W_pred, b_pred):
    raise NotImplementedError("write your pallas kernel here")


# speedup vs baseline: 5.2470x; 5.2470x over previous
import functools

import jax
import jax.numpy as jnp
import numpy as np
from jax import lax
from jax.experimental import pallas as pl
from jax.experimental.pallas import tpu as pltpu
from jax.experimental.pallas import tpu_sc as plsc

N = 10000
E = 320000
D = 128
DE = 16
G = 128
TASKS = 1

NP = 10240
NPT = 10248
TRASH = NP
STRIPE = 640
DEGC = 16
NC, NS = 2, 16
NW = NC * NS
EW = 10112
EP = EW * NW
CH = 128
NCH = EW // CH
ROWS80 = NP // 128
INT_MIN = np.int32(-2147483648)


def _msg_body(h_hbm, src_hbm, dst_hbm, eaw_hbm, kept_hbm, zrow_hbm,
              onerow_hbm, outh_hbm, outed_hbm, outdeg_hbm,
              kchunk_v, src_v, dst_v, dsc_v, rows_v, sem, semk, acc_sh):
    c = lax.axis_index("c")
    s = lax.axis_index("s")
    wid = s * NC + c
    off = s * STRIPE
    ebase = wid * EW

    def _zero_acc():
        pltpu.sync_copy(zrow_hbm, rows_v)
        for q in range(STRIPE // CH):
            pltpu.sync_copy(rows_v, acc_sh.at[pl.ds(off + q * CH, CH)])

    def _dump_acc(out_hbm):
        for q in range(STRIPE // CH):
            pltpu.sync_copy(acc_sh.at[pl.ds(off + q * CH, CH)], rows_v)
            pltpu.sync_copy(rows_v,
                            out_hbm.at[pl.ds(c * NPT + off + q * CH, CH)])

    _zero_acc()
    plsc.subcore_barrier()

    def _chunk1(j, carry):
        base = ebase + j * CH
        pltpu.sync_copy(src_hbm.at[pl.ds(base, CH)], src_v)
        pltpu.sync_copy(dst_hbm.at[pl.ds(base, CH)], dst_v)
        pltpu.async_copy(h_hbm.at[src_v], rows_v, sem).wait()
        pltpu.sync_copy(rows_v, acc_sh.at[dst_v], add=True)
        return carry

    lax.fori_loop(0, NCH, _chunk1, 0)
    plsc.subcore_barrier()
    _dump_acc(outh_hbm)

    _zero_acc()
    plsc.subcore_barrier()

    def _chunk2(j, carry):
        base = ebase + j * CH
        pltpu.sync_copy(src_hbm.at[pl.ds(base, CH)], src_v)
        pltpu.sync_copy(dst_hbm.at[pl.ds(base, CH)], dst_v)
        kg = pltpu.async_copy(kept_hbm.at[src_v], kchunk_v, semk)
        pltpu.sync_copy(eaw_hbm.at[pl.ds(base, CH)], rows_v)
        kg.wait()
        for t in range(CH // 16):
            kv = kchunk_v[pl.ds(t * 16, 16)]
            dv = dst_v[pl.ds(t * 16, 16)]
            dsc_v[pl.ds(t * 16, 16)] = jnp.where(
                kv > 0, dv, jnp.full((16,), TRASH, jnp.int32))
        pltpu.sync_copy(rows_v, acc_sh.at[dsc_v], add=True)
        return carry

    lax.fori_loop(0, NCH, _chunk2, 0)
    plsc.subcore_barrier()
    _dump_acc(outed_hbm)

    _zero_acc()
    plsc.subcore_barrier()

    def _chunk3(j, carry):
        base = ebase + j * CH
        pltpu.sync_copy(src_hbm.at[pl.ds(base, CH)], src_v)
        pltpu.sync_copy(dst_hbm.at[pl.ds(base, CH)], dst_v)
        kg = pltpu.async_copy(kept_hbm.at[src_v], kchunk_v, semk)
        pltpu.sync_copy(onerow_hbm, rows_v)
        kg.wait()
        for t in range(CH // 16):
            kv = kchunk_v[pl.ds(t * 16, 16)]
            dv = dst_v[pl.ds(t * 16, 16)]
            dsc_v[pl.ds(t * 16, 16)] = jnp.where(
                kv > 0, dv, jnp.full((16,), TRASH, jnp.int32))
        pltpu.sync_copy(rows_v, acc_sh.at[dsc_v], add=True)
        return carry

    lax.fori_loop(0, NCH, _chunk3, 0)
    plsc.subcore_barrier()
    _dump_acc(outdeg_hbm)


@functools.lru_cache(maxsize=None)
def _get_msg_call():
    mesh = plsc.VectorSubcoreMesh(
        core_axis_name="c", subcore_axis_name="s",
        num_cores=NC, num_subcores=NS)
    return pl.kernel(
        _msg_body,
        out_type=[
            jax.ShapeDtypeStruct((NC * NPT, D), jnp.float32),
            jax.ShapeDtypeStruct((NC * NPT, D), jnp.float32),
            jax.ShapeDtypeStruct((NC * NPT, D), jnp.float32),
        ],
        mesh=mesh,
        scratch_types=[
            pltpu.VMEM((CH,), jnp.int32),
            pltpu.VMEM((CH,), jnp.int32),
            pltpu.VMEM((CH,), jnp.int32),
            pltpu.VMEM((CH,), jnp.int32),
            pltpu.VMEM((CH, D), jnp.float32),
            pltpu.SemaphoreType.DMA,
            pltpu.SemaphoreType.DMA,
            pltpu.VMEM_SHARED((NPT, D), jnp.float32),
        ],
    )


EB = 4096


def _eaw_body(ea_ref, we_ref, o_ref):
    o_ref[...] = jnp.dot(ea_ref[...], we_ref[...],
                         preferred_element_type=jnp.float32)


_eaw_call = pl.pallas_call(
    _eaw_body,
    grid=(EP // EB,),
    in_specs=[pl.BlockSpec((EB, DE), lambda i: (i, 0)),
              pl.BlockSpec((DE, D), lambda i: (0, 0))],
    out_specs=pl.BlockSpec((EB, D), lambda i: (i, 0)),
    out_shape=jax.ShapeDtypeStruct((EP, D), jnp.float32))


def _enc_body(x_ref, w_ref, o_ref):
    o_ref[...] = jnp.dot(x_ref[...], w_ref[...],
                         preferred_element_type=jnp.float32)


_enc_call = pl.pallas_call(
    _enc_body, out_shape=jax.ShapeDtypeStruct((NP, D), jnp.float32))


def _mkkey(score):
    bits = lax.bitcast_convert_type(score, jnp.int32)
    return jnp.where(bits < 0, (~bits) ^ INT_MIN, bits)


def _radix_kth(keys, k):
    def it(i, cur):
        trial = cur + lax.shift_left(jnp.int32(1), 31 - i)
        cnt = jnp.sum((keys >= trial).astype(jnp.int32))
        return jnp.where(cnt >= k, trial, cur)

    return lax.fori_loop(0, 32, it, INT_MIN)


def _dense_h(aggh_ref, eaw_ref, deg_ref, wg_ref, bg_ref):
    aggh = aggh_ref[0:NP, :] + aggh_ref[NPT:NPT + NP, :]
    eaw = eaw_ref[0:NP, :] + eaw_ref[NPT:NPT + NP, :]
    deg = jnp.maximum(deg_ref[0:NP, :] + deg_ref[NPT:NPT + NP, :], 1.0)
    hmid = (aggh + eaw) / deg
    return jnp.maximum(
        jnp.dot(hmid, wg_ref[...], preferred_element_type=jnp.float32)
        + bg_ref[...], 0.0)


def _readout(batch_ref, hg):
    oh = (batch_ref[...] == lax.broadcasted_iota(jnp.int32, (NP, G), 1)
          ).astype(jnp.float32)
    return lax.dot_general(oh, hg, (((0,), (0,)), ((), ())),
                           preferred_element_type=jnp.float32)


def _stage_a_body(aggh_ref, eaw_ref, deg_ref, wg_ref, bg_ref, p_ref,
                  hn_ref, sc_ref):
    hn = _dense_h(aggh_ref, eaw_ref, deg_ref, wg_ref, bg_ref)
    pn = jnp.sqrt(jnp.sum(p_ref[...] * p_ref[...])) + 1e-8
    hn_ref[...] = hn
    sc_ref[...] = jnp.dot(hn, p_ref[...],
                          preferred_element_type=jnp.float32) / pn


_stage_a = pl.pallas_call(
    _stage_a_body,
    out_shape=[
        jax.ShapeDtypeStruct((NP, D), jnp.float32),
        jax.ShapeDtypeStruct((NP, 1), jnp.float32),
    ])


def _stage_b_body(k, hn_ref, sc_ref, sc80_ref, kept_ref, kept80_ref,
                  batch_ref, r_ref, hg_ref, keptn_ref, rn_ref):
    keys80 = jnp.where(kept80_ref[...] > 0, _mkkey(sc80_ref[...]), INT_MIN)
    thr = _radix_kth(keys80, k)
    score = sc_ref[...]
    alive = kept_ref[...] > 0
    keys = jnp.where(alive, _mkkey(score), INT_MIN)
    keptn = alive & (keys >= thr)
    gate = jnp.where(keptn, jnp.tanh(score), 0.0)
    hg = hn_ref[...] * gate
    hg_ref[...] = hg
    keptn_ref[...] = keptn.astype(jnp.int32)
    rn_ref[...] = r_ref[...] + _readout(batch_ref, hg)


def _stage_b(k):
    return pl.pallas_call(
        functools.partial(_stage_b_body, k),
        out_shape=[
            jax.ShapeDtypeStruct((NP, D), jnp.float32),
            jax.ShapeDtypeStruct((NP, 1), jnp.int32),
            jax.ShapeDtypeStruct((G, D), jnp.float32),
        ])


_stageb0 = _stage_b(5000)
_stageb1 = _stage_b(2500)


def _final_body(aggh_ref, eaw_ref, deg_ref, wg_ref, bg_ref, batch_ref,
                kept_ref, r_ref, wp_ref, bp_ref, o_ref):
    hn = _dense_h(aggh_ref, eaw_ref, deg_ref, wg_ref, bg_ref)
    gate = (kept_ref[...] > 0).astype(jnp.float32)
    hg = hn * gate
    r = r_ref[...] + _readout(batch_ref, hg)
    o_ref[...] = jnp.dot(r, wp_ref[...],
                         preferred_element_type=jnp.float32) + bp_ref[...]


_final_call = pl.pallas_call(
    _final_body, out_shape=jax.ShapeDtypeStruct((G, TASKS), jnp.float32))


def kernel(x, edge_index, edge_attr, batch, W_enc, Wg0, We0, bg0, Wg1, We1,
           bg1, Wg2, We2, bg2, p0, p1, W_pred, b_pred):
    f32 = jnp.float32
    xp = jnp.zeros((NP, D), f32).at[:N].set(x)
    srcp = jnp.zeros((EP,), jnp.int32).at[:E].set(edge_index[0])
    dstp = jnp.full((EP,), TRASH, jnp.int32).at[:E].set(edge_index[1])
    eap = jnp.zeros((EP, DE), f32).at[:E].set(edge_attr)
    onerow = jnp.zeros((CH, D), f32).at[:, 0].set(1.0)
    batch2 = jnp.zeros((NP, 1), jnp.int32).at[:N, 0].set(batch)
    kept = jnp.zeros((NP,), jnp.int32).at[:N].set(1)
    r = jnp.zeros((G, D), f32)
    zrow = jnp.zeros((CH, D), f32)

    h = _enc_call(xp, W_enc)
    bgs = [bg0.reshape(1, D), bg1.reshape(1, D), bg2.reshape(1, D)]
    wgs = [Wg0, Wg1, Wg2]
    wes = [We0, We1, We2]
    stages = [_stageb0, _stageb1]
    ps = [p0, p1]
    msg_call = _get_msg_call()
    for l in range(2):
        eaWp = eap @ wes[l]
        outh, outed, outdeg = msg_call(h, srcp, dstp, eaWp, kept, zrow,
                                       onerow)
        hn, sc = _stage_a(outh, outed, outdeg[:, 0:1], wgs[l], bgs[l],
                          ps[l].reshape(D, 1))
        h, keptn, r = stages[l](hn, sc, sc.reshape(ROWS80, 128),
                                kept.reshape(NP, 1),
                                kept.reshape(ROWS80, 128), batch2, r)
        kept = keptn.reshape(NP)
    eaWp = eap @ wes[2]
    outh, outed, outdeg = msg_call(h, srcp, dstp, eaWp, kept, zrow, onerow)
    return _final_call(outh, outed, outdeg[:, 0:1], wgs[2], bgs[2], batch2,
                       kept.reshape(NP, 1), r,
                       W_pred, b_pred.reshape(1, TASKS))
